# initial kernel scaffold (unmeasured)
import jax
import jax.numpy as jnp
from jax import lax
from jax.experimental import pallas as pl
from jax.experimental.pallas import tpu as pltpu

N_DEV = 8
SQ = 2048
SKV = 2048
D_MODEL = 1024
DH = 128
HEADS_PER_SHARD = 8
CHUNK = SQ // N_DEV
ROW_BLK = 1024
SCALE = 0.08838834764831843


def _body(x_ref, wq_ref, k_ref, v_ref, wo_ref, out_ref,
          rs_send, rs_recv, ag_send, ag_recv,
          rs_ssem, rs_rsem, ag_ssem, ag_rsem):
    my = lax.axis_index("i")
    right = (my + 1) % N_DEV
    left = (my + N_DEV - 1) % N_DEV

    barrier_sem = pltpu.get_barrier_semaphore()
    for nbr in (left, right):
        pl.semaphore_signal(barrier_sem, inc=1, device_id=(nbr,),
                            device_id_type=pl.DeviceIdType.MESH)
    pl.semaphore_wait(barrier_sem, 2)

    kb = lax.broadcasted_iota(jnp.int32, (1, SKV), 1) // 64
    for j in range(HEADS_PER_SHARD):
        c0, c1 = j * DH, (j + 1) * DH
        wq_j = wq_ref[:, c0:c1]
        k_j = k_ref[:, c0:c1]
        v_j = v_ref[:, c0:c1]
        wo_j = wo_ref[c0:c1, :]
        for r in range(SQ // ROW_BLK):
            rows = pl.ds(r * ROW_BLK, ROW_BLK)
            q = jnp.dot(x_ref[rows, :], wq_j,
                        preferred_element_type=jnp.float32)
            s = lax.dot_general(
                q, k_j, (((1,), (1,)), ((), ())),
                preferred_element_type=jnp.float32) * SCALE
            qb = (r * ROW_BLK
                  + lax.broadcasted_iota(jnp.int32, (ROW_BLK, 1), 0)) // 64
            mask = (qb == kb) | (kb == 0) | ((qb + kb) % 3 == 0)
            s = jnp.where(mask, s, -1e9)
            m = jnp.max(s, axis=-1, keepdims=True)
            w = jnp.exp(s - m)
            w = w / jnp.sum(w, axis=-1, keepdims=True)
            ctx = jnp.dot(w, v_j, preferred_element_type=jnp.float32)
            part = jnp.dot(ctx, wo_j, preferred_element_type=jnp.float32)
            if j == 0:
                out_ref[rows, :] = part
            else:
                out_ref[rows, :] = out_ref[rows, :] + part

    for h in range(N_DEV - 1):
        send_c = (my - h) % N_DEV
        recv_c = (my - h - 1) % N_DEV
        rs_send[h] = out_ref[pl.ds(send_c * CHUNK, CHUNK), :]
        rdma = pltpu.make_async_remote_copy(
            src_ref=rs_send.at[h],
            dst_ref=rs_recv.at[h],
            send_sem=rs_ssem.at[h],
            recv_sem=rs_rsem.at[h],
            device_id=(right,),
            device_id_type=pl.DeviceIdType.MESH,
        )
        rdma.start()
        rdma.wait()
        out_ref[pl.ds(recv_c * CHUNK, CHUNK), :] = (
            out_ref[pl.ds(recv_c * CHUNK, CHUNK), :] + rs_recv[h])

    ag_send[...] = out_ref[pl.ds(((my + 1) % N_DEV) * CHUNK, CHUNK), :]
    for h in range(N_DEV - 1):
        src = ag_send if h == 0 else ag_recv.at[h - 1]
        rdma = pltpu.make_async_remote_copy(
            src_ref=src,
            dst_ref=ag_recv.at[h],
            send_sem=ag_ssem.at[h],
            recv_sem=ag_rsem.at[h],
            device_id=(right,),
            device_id_type=pl.DeviceIdType.MESH,
        )
        rdma.start()
        rdma.wait()
        out_ref[pl.ds(((my - h) % N_DEV) * CHUNK, CHUNK), :] = ag_recv[h]


def kernel(x, Wq, K_ext, V_ext, Wo):
    pos = lax.axis_index("i")
    x2 = x.reshape(SQ, D_MODEL)
    Ks = lax.dynamic_slice_in_dim(
        K_ext, pos * HEADS_PER_SHARD, HEADS_PER_SHARD, axis=2
    ).reshape(SKV, HEADS_PER_SHARD * DH)
    Vs = lax.dynamic_slice_in_dim(
        V_ext, pos * HEADS_PER_SHARD, HEADS_PER_SHARD, axis=2
    ).reshape(SKV, HEADS_PER_SHARD * DH)

    out = pl.pallas_call(
        _body,
        out_shape=jax.ShapeDtypeStruct((SQ, D_MODEL), jnp.float32),
        in_specs=[pl.BlockSpec(memory_space=pltpu.VMEM)] * 5,
        out_specs=pl.BlockSpec(memory_space=pltpu.VMEM),
        scratch_shapes=[
            pltpu.VMEM((N_DEV - 1, CHUNK, D_MODEL), jnp.float32),
            pltpu.VMEM((N_DEV - 1, CHUNK, D_MODEL), jnp.float32),
            pltpu.VMEM((CHUNK, D_MODEL), jnp.float32),
            pltpu.VMEM((N_DEV - 1, CHUNK, D_MODEL), jnp.float32),
            pltpu.SemaphoreType.DMA((N_DEV - 1,)),
            pltpu.SemaphoreType.DMA((N_DEV - 1,)),
            pltpu.SemaphoreType.DMA((N_DEV - 1,)),
            pltpu.SemaphoreType.DMA((N_DEV - 1,)),
        ],
        compiler_params=pltpu.CompilerParams(collective_id=0),
    )(x2, Wq, Ks, Vs, Wo)
    return out.reshape(1, SQ, D_MODEL)


# baseline (device time: 384990 ns/iter reference)
import jax
import jax.numpy as jnp
from jax import lax
from jax.experimental import pallas as pl
from jax.experimental.pallas import tpu as pltpu

N_DEV = 8
SQ = 2048
SKV = 2048
D_MODEL = 1024
DH = 128
HEADS_PER_SHARD = 8
CHUNK = SQ // N_DEV
ROW_BLK = 256
SCALE = 0.08838834764831843


def _body(x_ref, wq_ref, k_hbm, v_hbm, wo_ref, out_ref,
          k_buf, v_buf, rs_recv, ag_recv,
          kv_sems, rs_ssem, rs_rsem, ag_ssem, ag_rsem):
    my = lax.axis_index("i")
    right = (my + 1) % N_DEV
    left = (my + N_DEV - 1) % N_DEV

    barrier_sem = pltpu.get_barrier_semaphore()
    for nbr in (left, right):
        pl.semaphore_signal(barrier_sem, inc=1, device_id=(nbr,),
                            device_id_type=pl.DeviceIdType.MESH)
    pl.semaphore_wait(barrier_sem, 2)

    kb = lax.broadcasted_iota(jnp.int32, (1, SKV), 1) // 64

    for j in range(HEADS_PER_SHARD):
        c0, c1 = j * DH, (j + 1) * DH
        ck = pltpu.make_async_copy(k_hbm.at[:, c0:c1], k_buf, kv_sems.at[0])
        cv = pltpu.make_async_copy(v_hbm.at[:, c0:c1], v_buf, kv_sems.at[1])
        ck.start()
        cv.start()
        ck.wait()
        cv.wait()
        wq_j = wq_ref[:, c0:c1]
        wo_j = wo_ref[c0:c1, :]
        k_j = k_buf[...]
        v_j = v_buf[...]
        for r in range(SQ // ROW_BLK):
            rows = pl.ds(r * ROW_BLK, ROW_BLK)
            q = jnp.dot(x_ref[rows, :], wq_j,
                        preferred_element_type=jnp.float32)
            s = lax.dot_general(
                q, k_j, (((1,), (1,)), ((), ())),
                preferred_element_type=jnp.float32) * SCALE
            qb = (r * ROW_BLK
                  + lax.broadcasted_iota(jnp.int32, (ROW_BLK, 1), 0)) // 64
            mask = (qb == kb) | (kb == 0) | ((qb + kb) % 3 == 0)
            s = jnp.where(mask, s, -1e9)
            m = jnp.max(s, axis=-1, keepdims=True)
            w = jnp.exp(s - m)
            w = w / jnp.sum(w, axis=-1, keepdims=True)
            ctx = jnp.dot(w, v_j, preferred_element_type=jnp.float32)
            part = jnp.dot(ctx, wo_j, preferred_element_type=jnp.float32)
            if j == 0:
                out_ref[rows, :] = part
            else:
                out_ref[rows, :] = out_ref[rows, :] + part

    for h in range(N_DEV - 1):
        send_c = (my - h) % N_DEV
        recv_c = (my - h - 1) % N_DEV
        rdma = pltpu.make_async_remote_copy(
            src_ref=out_ref.at[pl.ds(send_c * CHUNK, CHUNK), :],
            dst_ref=rs_recv.at[h],
            send_sem=rs_ssem.at[h],
            recv_sem=rs_rsem.at[h],
            device_id=(right,),
            device_id_type=pl.DeviceIdType.MESH,
        )
        rdma.start()
        rdma.wait()
        out_ref[pl.ds(recv_c * CHUNK, CHUNK), :] = (
            out_ref[pl.ds(recv_c * CHUNK, CHUNK), :] + rs_recv[h])

    for h in range(N_DEV - 1):
        src = (out_ref.at[pl.ds(((my + 1) % N_DEV) * CHUNK, CHUNK), :]
               if h == 0 else ag_recv.at[h - 1])
        rdma = pltpu.make_async_remote_copy(
            src_ref=src,
            dst_ref=ag_recv.at[h],
            send_sem=ag_ssem.at[h],
            recv_sem=ag_rsem.at[h],
            device_id=(right,),
            device_id_type=pl.DeviceIdType.MESH,
        )
        rdma.start()
        rdma.wait()
        out_ref[pl.ds(((my - h) % N_DEV) * CHUNK, CHUNK), :] = ag_recv[h]


def kernel(x, Wq, K_ext, V_ext, Wo):
    pos = lax.axis_index("i")
    x2 = x.reshape(SQ, D_MODEL)
    Ks = lax.dynamic_slice_in_dim(
        K_ext, pos * HEADS_PER_SHARD, HEADS_PER_SHARD, axis=2
    ).reshape(SKV, HEADS_PER_SHARD * DH)
    Vs = lax.dynamic_slice_in_dim(
        V_ext, pos * HEADS_PER_SHARD, HEADS_PER_SHARD, axis=2
    ).reshape(SKV, HEADS_PER_SHARD * DH)

    out = pl.pallas_call(
        _body,
        out_shape=jax.ShapeDtypeStruct((SQ, D_MODEL), jnp.float32),
        in_specs=[
            pl.BlockSpec(memory_space=pltpu.VMEM),
            pl.BlockSpec(memory_space=pltpu.VMEM),
            pl.BlockSpec(memory_space=pltpu.MemorySpace.HBM),
            pl.BlockSpec(memory_space=pltpu.MemorySpace.HBM),
            pl.BlockSpec(memory_space=pltpu.VMEM),
        ],
        out_specs=pl.BlockSpec(memory_space=pltpu.VMEM),
        scratch_shapes=[
            pltpu.VMEM((SKV, DH), jnp.float32),
            pltpu.VMEM((SKV, DH), jnp.float32),
            pltpu.VMEM((N_DEV - 1, CHUNK, D_MODEL), jnp.float32),
            pltpu.VMEM((N_DEV - 1, CHUNK, D_MODEL), jnp.float32),
            pltpu.SemaphoreType.DMA((2,)),
            pltpu.SemaphoreType.DMA((N_DEV - 1,)),
            pltpu.SemaphoreType.DMA((N_DEV - 1,)),
            pltpu.SemaphoreType.DMA((N_DEV - 1,)),
            pltpu.SemaphoreType.DMA((N_DEV - 1,)),
        ],
        compiler_params=pltpu.CompilerParams(
            collective_id=0,
            vmem_limit_bytes=63 * 1024 * 1024,
        ),
    )(x2, Wq, Ks, Vs, Wo)
    return out.reshape(1, SQ, D_MODEL)
